# baseline (speedup 1.0000x reference)
import jax
import jax.numpy as jnp
from jax import lax
from jax.experimental import pallas as pl
from jax.experimental.pallas import tpu as pltpu
from jax.experimental.pallas import tpu_sc as plsc

TOD, DOW, DOM, DOY = 288, 7, 31, 366
D = 64
B = 4096
S = 200
NC, NS = 2, 16
NW = NC * NS
NB = B // NW
R_DD = DOW * DOM
ROWS = TOD + R_DD + DOY
SP = D + 1


def _sc_body(x_hbm, tab_hbm, out_hbm, tab_v, x_v, out_v, sx, so):
    wid = lax.axis_index("s") * NC + lax.axis_index("c")
    b0 = wid * NB
    pltpu.sync_copy(tab_hbm, tab_v)

    out_v2 = out_v

    pltpu.async_copy(x_hbm.at[0, :, pl.ds(b0, NB)], x_v[0], sx[0])

    def compute_chunk(x_vp, out_vp):
        @plsc.parallel_loop(0, NB // 16, unroll=4)
        def group_body(g):
            xv0 = x_vp[0, pl.ds(16 * g, 16)]
            xv1 = x_vp[1, pl.ds(16 * g, 16)]
            xv2 = x_vp[2, pl.ds(16 * g, 16)]
            xv3 = x_vp[3, pl.ds(16 * g, 16)]
            rv0 = ((xv0 + 0.5) * float(TOD)).astype(jnp.int32) * SP
            i1 = ((xv1 + 0.5) * float(DOW)).astype(jnp.int32)
            i2 = ((xv2 + 0.5) * float(DOM)).astype(jnp.int32)
            rv1 = i1 * (DOM * SP) + i2 * SP + TOD * SP
            rv2 = (((xv3 + 0.5) * float(DOY)).astype(jnp.int32) * SP
                   + (TOD + R_DD) * SP)
            for d in range(D):
                v = (plsc.load_gather(tab_v, [rv0 + d])
                     + plsc.load_gather(tab_v, [rv1 + d])
                     + plsc.load_gather(tab_v, [rv2 + d]))
                out_vp[d, pl.ds(16 * g, 16)] = v

    def s_pair(sj, carry):
        for p in range(2):
            s = 2 * sj + p
            pltpu.make_async_copy(
                x_hbm.at[s, :, pl.ds(b0, NB)], x_v[p], sx[p]).wait()

            @pl.when(s + 1 < S)
            def _():
                pltpu.async_copy(
                    x_hbm.at[s + 1, :, pl.ds(b0, NB)], x_v[1 - p], sx[1 - p])

            @pl.when(s >= 2)
            def _():
                pltpu.make_async_copy(
                    out_v2[p], out_hbm.at[s - 2, :, pl.ds(b0, NB)],
                    so[p]).wait()

            compute_chunk(x_v[p], out_v[p])
            pltpu.async_copy(
                out_v2[p], out_hbm.at[s, :, pl.ds(b0, NB)], so[p])
        return carry

    lax.fori_loop(0, S // 2, s_pair, 0)
    for p in range(2):
        pltpu.make_async_copy(
            out_v2[p], out_hbm.at[S - 2 + p, :, pl.ds(b0, NB)], so[p]).wait()


def kernel(x, w_tod, w_dow, w_dom, w_doy):
    w_dd = (w_dow[:, None, :] + w_dom[None, :, :]).reshape(R_DD, D)
    tab = jnp.concatenate([w_tod, w_dd, w_doy], axis=0)
    tab = jnp.pad(tab, ((0, 0), (0, SP - D))).reshape(-1)
    xt = x.transpose(1, 2, 0)
    mesh = plsc.VectorSubcoreMesh(core_axis_name="c", subcore_axis_name="s")
    out = pl.kernel(
        _sc_body,
        out_type=jax.ShapeDtypeStruct((S, D, B), jnp.float32),
        mesh=mesh,
        scratch_types=[
            pltpu.VMEM((ROWS * SP,), jnp.float32),
            [pltpu.VMEM((4, NB), jnp.float32)] * 2,
            [pltpu.VMEM((D, NB), jnp.float32)] * 2,
            [pltpu.SemaphoreType.DMA] * 2,
            [pltpu.SemaphoreType.DMA] * 2,
        ],
        compiler_params=pltpu.CompilerParams(needs_layout_passes=False),
    )(xt, tab)
    return out.transpose(2, 0, 1)


# speedup vs baseline: 2.2986x; 2.2986x over previous
import jax
import jax.numpy as jnp
from jax import lax
from jax.experimental import pallas as pl
from jax.experimental.pallas import tpu as pltpu
from jax.experimental.pallas import tpu_sc as plsc

TOD, DOW, DOM, DOY = 288, 7, 31, 366
D = 64
B = 4096
S = 200
NC, NS = 2, 16
NW = NC * NS
NB = B // NW
R_DD = DOW * DOM
ROWS = TOD + R_DD + DOY
SP = D + 1


def _sc_body(x_hbm, tab_hbm, out_hbm, tab_v, x_v, out_v, sx, so):
    wid = lax.axis_index("s") * NC + lax.axis_index("c")
    b0 = wid * NB
    pltpu.sync_copy(tab_hbm, tab_v)

    out_v2 = out_v

    pltpu.async_copy(x_hbm.at[0, :, pl.ds(b0, NB)], x_v[0], sx[0])

    def compute_chunk(x_vp, out_vp):
        @plsc.parallel_loop(0, NB // 16, unroll=2)
        def group_body(g):
            xv0 = x_vp[0, pl.ds(16 * g, 16)]
            xv1 = x_vp[1, pl.ds(16 * g, 16)]
            xv2 = x_vp[2, pl.ds(16 * g, 16)]
            xv3 = x_vp[3, pl.ds(16 * g, 16)]
            rv0 = ((xv0 + 0.5) * float(TOD)).astype(jnp.int32) * SP
            i1 = ((xv1 + 0.5) * float(DOW)).astype(jnp.int32)
            i2 = ((xv2 + 0.5) * float(DOM)).astype(jnp.int32)
            rv1 = i1 * (DOM * SP) + i2 * SP + TOD * SP
            rv2 = (((xv3 + 0.5) * float(DOY)).astype(jnp.int32) * SP
                   + (TOD + R_DD) * SP)
            for d in range(D):
                v = (plsc.load_gather(tab_v, [rv0 + d])
                     + plsc.load_gather(tab_v, [rv1 + d])
                     + plsc.load_gather(tab_v, [rv2 + d]))
                out_vp[d, pl.ds(16 * g, 16)] = v

    def s_pair(sj, carry):
        for p in range(2):
            s = 2 * sj + p
            pltpu.make_async_copy(
                x_hbm.at[s, :, pl.ds(b0, NB)], x_v[p], sx[p]).wait()

            @pl.when(s + 1 < S)
            def _():
                pltpu.async_copy(
                    x_hbm.at[s + 1, :, pl.ds(b0, NB)], x_v[1 - p], sx[1 - p])

            @pl.when(s >= 2)
            def _():
                pltpu.make_async_copy(
                    out_v2[p], out_hbm.at[s - 2, :, pl.ds(b0, NB)],
                    so[p]).wait()

            compute_chunk(x_v[p], out_v[p])
            pltpu.async_copy(
                out_v2[p], out_hbm.at[s, :, pl.ds(b0, NB)], so[p])
        return carry

    lax.fori_loop(0, S // 2, s_pair, 0)
    for p in range(2):
        pltpu.make_async_copy(
            out_v2[p], out_hbm.at[S - 2 + p, :, pl.ds(b0, NB)], so[p]).wait()


def kernel(x, w_tod, w_dow, w_dom, w_doy):
    w_dd = (w_dow[:, None, :] + w_dom[None, :, :]).reshape(R_DD, D)
    tab = jnp.concatenate([w_tod, w_dd, w_doy], axis=0)
    tab = jnp.pad(tab, ((0, 0), (0, SP - D))).reshape(-1)
    xt = x.transpose(1, 2, 0)
    mesh = plsc.VectorSubcoreMesh(core_axis_name="c", subcore_axis_name="s")
    out = pl.kernel(
        _sc_body,
        out_type=jax.ShapeDtypeStruct((S, D, B), jnp.float32),
        mesh=mesh,
        scratch_types=[
            pltpu.VMEM((ROWS * SP,), jnp.float32),
            [pltpu.VMEM((4, NB), jnp.float32)] * 2,
            [pltpu.VMEM((D, NB), jnp.float32)] * 2,
            [pltpu.SemaphoreType.DMA] * 2,
            [pltpu.SemaphoreType.DMA] * 2,
        ],
        compiler_params=pltpu.CompilerParams(needs_layout_passes=False),
    )(xt, tab)
    return out.transpose(2, 0, 1)


# speedup vs baseline: 4.5985x; 2.0006x over previous
import jax
import jax.numpy as jnp
from jax import lax
from jax.experimental import pallas as pl
from jax.experimental.pallas import tpu as pltpu
from jax.experimental.pallas import tpu_sc as plsc

TOD, DOW, DOM, DOY = 288, 7, 31, 366
D = 64
B = 4096
S = 200
NC, NS = 2, 16
NW = NC * NS
NB = B // NW
R_DD = DOW * DOM
ROWS = TOD + R_DD + DOY
DP = D // 2
SP = DP + 1


def _sc_body(x_hbm, tab_hbm, out_hbm, tab_v, x_v, out_v, sx, so):
    wid = lax.axis_index("s") * NC + lax.axis_index("c")
    b0 = wid * NB
    pltpu.sync_copy(tab_hbm, tab_v)
    pltpu.async_copy(x_hbm.at[0, :, pl.ds(b0, NB)], x_v[0], sx[0])

    def compute_chunk(x_vp, out_vp):
        @plsc.parallel_loop(0, NB // 16, unroll=2)
        def group_body(g):
            xv0 = x_vp[0, pl.ds(16 * g, 16)]
            xv1 = x_vp[1, pl.ds(16 * g, 16)]
            xv2 = x_vp[2, pl.ds(16 * g, 16)]
            xv3 = x_vp[3, pl.ds(16 * g, 16)]
            rv0 = ((xv0 + 0.5) * float(TOD)).astype(jnp.int32) * SP
            i1 = ((xv1 + 0.5) * float(DOW)).astype(jnp.int32)
            i2 = ((xv2 + 0.5) * float(DOM)).astype(jnp.int32)
            rv1 = i1 * (DOM * SP) + i2 * SP + TOD * SP
            rv2 = (((xv3 + 0.5) * float(DOY)).astype(jnp.int32) * SP
                   + (TOD + R_DD) * SP)
            for p in range(DP):
                w0 = plsc.load_gather(tab_v, [rv0 + p])
                w1 = plsc.load_gather(tab_v, [rv1 + p])
                w2 = plsc.load_gather(tab_v, [rv2 + p])
                e0, o0 = plsc.unpack(plsc.bitcast(w0, jnp.bfloat16),
                                     format=plsc.PackFormat.INTERLEAVED)
                e1, o1 = plsc.unpack(plsc.bitcast(w1, jnp.bfloat16),
                                     format=plsc.PackFormat.INTERLEAVED)
                e2, o2 = plsc.unpack(plsc.bitcast(w2, jnp.bfloat16),
                                     format=plsc.PackFormat.INTERLEAVED)
                out_vp[2 * p, pl.ds(16 * g, 16)] = e0 + e1 + e2
                out_vp[2 * p + 1, pl.ds(16 * g, 16)] = o0 + o1 + o2

    def s_pair(sj, carry):
        for p in range(2):
            s = 2 * sj + p
            pltpu.make_async_copy(
                x_hbm.at[s, :, pl.ds(b0, NB)], x_v[p], sx[p]).wait()

            @pl.when(s + 1 < S)
            def _():
                pltpu.async_copy(
                    x_hbm.at[s + 1, :, pl.ds(b0, NB)], x_v[1 - p], sx[1 - p])

            @pl.when(s >= 2)
            def _():
                pltpu.make_async_copy(
                    out_v[p], out_hbm.at[s - 2, :, pl.ds(b0, NB)],
                    so[p]).wait()

            compute_chunk(x_v[p], out_v[p])
            pltpu.async_copy(
                out_v[p], out_hbm.at[s, :, pl.ds(b0, NB)], so[p])
        return carry

    lax.fori_loop(0, S // 2, s_pair, 0)
    for p in range(2):
        pltpu.make_async_copy(
            out_v[p], out_hbm.at[S - 2 + p, :, pl.ds(b0, NB)], so[p]).wait()


def kernel(x, w_tod, w_dow, w_dom, w_doy):
    w_dd = (w_dow[:, None, :] + w_dom[None, :, :]).reshape(R_DD, D)
    tab = jnp.concatenate([w_tod, w_dd, w_doy], axis=0)
    t16 = lax.bitcast_convert_type(tab.astype(jnp.bfloat16), jnp.uint16)
    packed = (t16[:, 0::2].astype(jnp.uint32)
              | (t16[:, 1::2].astype(jnp.uint32) << 16))
    packed = jnp.pad(packed, ((0, 0), (0, SP - DP)))
    tabw = lax.bitcast_convert_type(packed, jnp.int32).reshape(-1)
    xt = x.transpose(1, 2, 0)
    mesh = plsc.VectorSubcoreMesh(core_axis_name="c", subcore_axis_name="s")
    out = pl.kernel(
        _sc_body,
        out_type=jax.ShapeDtypeStruct((S, D, B), jnp.float32),
        mesh=mesh,
        scratch_types=[
            pltpu.VMEM((ROWS * SP,), jnp.int32),
            [pltpu.VMEM((4, NB), jnp.float32)] * 2,
            [pltpu.VMEM((D, NB), jnp.float32)] * 2,
            [pltpu.SemaphoreType.DMA] * 2,
            [pltpu.SemaphoreType.DMA] * 2,
        ],
        compiler_params=pltpu.CompilerParams(needs_layout_passes=False),
    )(xt, tabw)
    return out.transpose(2, 0, 1)
